# baseline (device time: 11200 ns/iter reference)
import jax
import jax.numpy as jnp
from jax import lax
from jax.experimental import pallas as pl
from jax.experimental.pallas import tpu as pltpu

N_DEV = 4
EPS = 1e-5
N_CHUNK = 4


def kernel(x, gamma, beta):
    m, n_per = x.shape
    n_total = N_DEV * n_per
    rows = m // N_CHUNK
    gamma2d = gamma.reshape(1, n_per)
    beta2d = beta.reshape(1, n_per)

    def body(x_hbm, g_ref, b_ref, out_hbm, x_vmem, out_vmem, stats_ref,
             comm_ref, send_sems, recv_sems, in_sems, out_sems):
        my_pos = lax.axis_index("i")

        in_copies = []
        for c in range(N_CHUNK):
            cp = pltpu.make_async_copy(
                x_hbm.at[pl.ds(c * rows, rows), :],
                x_vmem.at[pl.ds(c * rows, rows), :],
                in_sems.at[c],
            )
            cp.start()
            in_copies.append(cp)

        barrier_sem = pltpu.get_barrier_semaphore()
        for k in range(1, N_DEV):
            peer = lax.rem(my_pos + k, N_DEV)
            pl.semaphore_signal(
                barrier_sem, inc=1,
                device_id=(peer,), device_id_type=pl.DeviceIdType.MESH,
            )

        for c in range(N_CHUNK):
            in_copies[c].wait()
            xc = x_vmem[pl.ds(c * rows, rows), :]
            stats_ref[0, pl.ds(c * rows, rows)] = jnp.sum(xc, axis=1)
            stats_ref[1, pl.ds(c * rows, rows)] = jnp.sum(xc * xc, axis=1)

        pl.semaphore_wait(barrier_sem, N_DEV - 1)

        rdmas = []
        for k in range(1, N_DEV):
            peer = lax.rem(my_pos + k, N_DEV)
            slot = N_DEV - 1 - k
            rdma = pltpu.make_async_remote_copy(
                src_ref=stats_ref,
                dst_ref=comm_ref.at[slot],
                send_sem=send_sems.at[k - 1],
                recv_sem=recv_sems.at[slot],
                device_id=(peer,),
                device_id_type=pl.DeviceIdType.MESH,
            )
            rdma.start()
            rdmas.append(rdma)

        gx = x_vmem[:, :] * g_ref[0, :]

        for rdma in rdmas:
            rdma.wait_send()
        for j in range(N_DEV - 1):
            recv = pltpu.make_async_remote_copy(
                src_ref=stats_ref,
                dst_ref=comm_ref.at[j],
                send_sem=send_sems.at[0],
                recv_sem=recv_sems.at[j],
                device_id=(my_pos,),
                device_id_type=pl.DeviceIdType.MESH,
            )
            recv.wait_recv()

        tot = (stats_ref[:, :] + comm_ref[0, :, :]
               + comm_ref[1, :, :] + comm_ref[2, :, :])
        inv_n = 1.0 / n_total
        mean_r = tot[0:1, :] * inv_n
        var_r = tot[1:2, :] * inv_n - mean_r * mean_r
        rstd_r = lax.rsqrt(var_r + EPS)
        mean_c = mean_r.reshape(m, 1)
        rstd_c = rstd_r.reshape(m, 1)

        out_copies = []
        for c in range(N_CHUNK):
            sl = slice(c * rows, (c + 1) * rows)
            out_vmem[pl.ds(c * rows, rows), :] = (
                (gx[sl, :] - mean_c[sl, :] * g_ref[0, :]) * rstd_c[sl, :]
                + b_ref[0, :]
            )
            cp = pltpu.make_async_copy(
                out_vmem.at[pl.ds(c * rows, rows), :],
                out_hbm.at[pl.ds(c * rows, rows), :],
                out_sems.at[c],
            )
            cp.start()
            out_copies.append(cp)
        for cp in out_copies:
            cp.wait()

    return pl.pallas_call(
        body,
        out_shape=jax.ShapeDtypeStruct((m, n_per), jnp.float32),
        in_specs=[
            pl.BlockSpec(memory_space=pl.ANY),
            pl.BlockSpec(memory_space=pltpu.VMEM),
            pl.BlockSpec(memory_space=pltpu.VMEM),
        ],
        out_specs=pl.BlockSpec(memory_space=pl.ANY),
        scratch_shapes=[
            pltpu.VMEM((m, n_per), jnp.float32),
            pltpu.VMEM((m, n_per), jnp.float32),
            pltpu.VMEM((2, m), jnp.float32),
            pltpu.VMEM((N_DEV - 1, 2, m), jnp.float32),
            pltpu.SemaphoreType.DMA((N_DEV - 1,)),
            pltpu.SemaphoreType.DMA((N_DEV - 1,)),
            pltpu.SemaphoreType.DMA((N_CHUNK,)),
            pltpu.SemaphoreType.DMA((N_CHUNK,)),
        ],
        compiler_params=pltpu.CompilerParams(collective_id=0),
    )(x, gamma2d, beta2d)


# device time: 10831 ns/iter; 1.0341x vs baseline; 1.0341x over previous
import jax
import jax.numpy as jnp
from jax import lax
from jax.experimental import pallas as pl
from jax.experimental.pallas import tpu as pltpu

N_DEV = 4
EPS = 1e-5
N_CHUNK = 2


def kernel(x, gamma, beta):
    m, n_per = x.shape
    n_total = N_DEV * n_per
    rows = m // N_CHUNK
    gamma2d = gamma.reshape(1, n_per)
    beta2d = beta.reshape(1, n_per)

    def body(x_ref, g_ref, b_ref, out_hbm, out_vmem, stats_ref,
             comm_ref, send_sems, recv_sems, out_sems):
        my_pos = lax.axis_index("i")

        barrier_sem = pltpu.get_barrier_semaphore()
        for k in range(1, N_DEV):
            peer = lax.rem(my_pos + k, N_DEV)
            pl.semaphore_signal(
                barrier_sem, inc=1,
                device_id=(peer,), device_id_type=pl.DeviceIdType.MESH,
            )

        xv = x_ref[:, :]
        stats_ref[0, :] = jnp.sum(xv, axis=1)
        stats_ref[1, :] = jnp.sum(xv * xv, axis=1)

        pl.semaphore_wait(barrier_sem, N_DEV - 1)

        rdmas = []
        for k in range(1, N_DEV):
            peer = lax.rem(my_pos + k, N_DEV)
            slot = N_DEV - 1 - k
            rdma = pltpu.make_async_remote_copy(
                src_ref=stats_ref,
                dst_ref=comm_ref.at[slot],
                send_sem=send_sems.at[k - 1],
                recv_sem=recv_sems.at[slot],
                device_id=(peer,),
                device_id_type=pl.DeviceIdType.MESH,
            )
            rdma.start()
            rdmas.append(rdma)
        for rdma in rdmas:
            rdma.wait_send()
        for j in range(N_DEV - 1):
            recv = pltpu.make_async_remote_copy(
                src_ref=stats_ref,
                dst_ref=comm_ref.at[j],
                send_sem=send_sems.at[0],
                recv_sem=recv_sems.at[j],
                device_id=(my_pos,),
                device_id_type=pl.DeviceIdType.MESH,
            )
            recv.wait_recv()

        tot = (stats_ref[:, :] + comm_ref[0, :, :]
               + comm_ref[1, :, :] + comm_ref[2, :, :])
        inv_n = 1.0 / n_total
        mean_r = tot[0:1, :] * inv_n
        var_r = tot[1:2, :] * inv_n - mean_r * mean_r
        rstd_r = lax.rsqrt(var_r + EPS)
        mean_c = mean_r.reshape(m, 1)
        rstd_c = rstd_r.reshape(m, 1)

        out_copies = []
        for c in range(N_CHUNK):
            sl = slice(c * rows, (c + 1) * rows)
            out_vmem[pl.ds(c * rows, rows), :] = (
                (xv[sl, :] - mean_c[sl, :]) * rstd_c[sl, :] * g_ref[0, :]
                + b_ref[0, :]
            )
            cp = pltpu.make_async_copy(
                out_vmem.at[pl.ds(c * rows, rows), :],
                out_hbm.at[pl.ds(c * rows, rows), :],
                out_sems.at[c],
            )
            cp.start()
            out_copies.append(cp)
        for cp in out_copies:
            cp.wait()

    return pl.pallas_call(
        body,
        out_shape=jax.ShapeDtypeStruct((m, n_per), jnp.float32),
        in_specs=[
            pl.BlockSpec(memory_space=pltpu.VMEM),
            pl.BlockSpec(memory_space=pltpu.VMEM),
            pl.BlockSpec(memory_space=pltpu.VMEM),
        ],
        out_specs=pl.BlockSpec(memory_space=pl.ANY),
        scratch_shapes=[
            pltpu.VMEM((m, n_per), jnp.float32),
            pltpu.VMEM((2, m), jnp.float32),
            pltpu.VMEM((N_DEV - 1, 2, m), jnp.float32),
            pltpu.SemaphoreType.DMA((N_DEV - 1,)),
            pltpu.SemaphoreType.DMA((N_DEV - 1,)),
            pltpu.SemaphoreType.DMA((N_CHUNK,)),
        ],
        compiler_params=pltpu.CompilerParams(collective_id=0),
    )(x, gamma2d, beta2d)


# device time: 10556 ns/iter; 1.0610x vs baseline; 1.0261x over previous
import jax
import jax.numpy as jnp
from jax import lax
from jax.experimental import pallas as pl
from jax.experimental.pallas import tpu as pltpu

N_DEV = 4
EPS = 1e-5


def kernel(x, gamma, beta):
    m, n_per = x.shape
    n_total = N_DEV * n_per
    gamma2d = gamma.reshape(1, n_per)
    beta2d = beta.reshape(1, n_per)

    def body(x_ref, g_ref, b_ref, out_ref, stats_ref, comm_ref,
             send_sems, recv_sems):
        my_pos = lax.axis_index("i")

        barrier_sem = pltpu.get_barrier_semaphore()
        for k in range(1, N_DEV):
            peer = lax.rem(my_pos + k, N_DEV)
            pl.semaphore_signal(
                barrier_sem, inc=1,
                device_id=(peer,), device_id_type=pl.DeviceIdType.MESH,
            )

        xv = x_ref[:, :]
        stats_ref[0, :] = jnp.sum(xv, axis=1)
        stats_ref[1, :] = jnp.sum(xv * xv, axis=1)

        pl.semaphore_wait(barrier_sem, N_DEV - 1)

        rdmas = []
        for k in range(1, N_DEV):
            peer = lax.rem(my_pos + k, N_DEV)
            slot = N_DEV - 1 - k
            rdma = pltpu.make_async_remote_copy(
                src_ref=stats_ref,
                dst_ref=comm_ref.at[slot],
                send_sem=send_sems.at[k - 1],
                recv_sem=recv_sems.at[slot],
                device_id=(peer,),
                device_id_type=pl.DeviceIdType.MESH,
            )
            rdma.start()
            rdmas.append(rdma)

        gx = xv * g_ref[0, :]

        for rdma in rdmas:
            rdma.wait_send()
        for j in range(N_DEV - 1):
            recv = pltpu.make_async_remote_copy(
                src_ref=stats_ref,
                dst_ref=comm_ref.at[j],
                send_sem=send_sems.at[0],
                recv_sem=recv_sems.at[j],
                device_id=(my_pos,),
                device_id_type=pl.DeviceIdType.MESH,
            )
            recv.wait_recv()

        tot = (stats_ref[:, :] + comm_ref[0, :, :]
               + comm_ref[1, :, :] + comm_ref[2, :, :])
        inv_n = 1.0 / n_total
        mean_r = tot[0:1, :] * inv_n
        var_r = tot[1:2, :] * inv_n - mean_r * mean_r
        rstd_r = lax.rsqrt(var_r + EPS)
        mean_c = mean_r.reshape(m, 1)
        rstd_c = rstd_r.reshape(m, 1)
        out_ref[:, :] = (gx - mean_c * g_ref[0, :]) * rstd_c + b_ref[0, :]

    return pl.pallas_call(
        body,
        out_shape=jax.ShapeDtypeStruct((m, n_per), jnp.float32),
        in_specs=[
            pl.BlockSpec(memory_space=pltpu.VMEM),
            pl.BlockSpec(memory_space=pltpu.VMEM),
            pl.BlockSpec(memory_space=pltpu.VMEM),
        ],
        out_specs=pl.BlockSpec(memory_space=pltpu.VMEM),
        scratch_shapes=[
            pltpu.VMEM((2, m), jnp.float32),
            pltpu.VMEM((N_DEV - 1, 2, m), jnp.float32),
            pltpu.SemaphoreType.DMA((N_DEV - 1,)),
            pltpu.SemaphoreType.DMA((N_DEV - 1,)),
        ],
        compiler_params=pltpu.CompilerParams(collective_id=0),
    )(x, gamma2d, beta2d)
